# baseline (speedup 1.0000x reference)
import jax
import jax.numpy as jnp
from jax.experimental import pallas as pl
from jax.experimental.pallas import tpu as pltpu
from jax.experimental.pallas import tpu_sc as plsc

B, N_NODE, W_LEN, DLEN = 128, 200, 10, 30
H = 512
F32 = jnp.float32
BF16 = jnp.bfloat16
BB = 2
GW = 128


def _sc_gather(table, idx_flat):
    v, hdim = table.shape
    t2 = table.reshape(2 * v, hdim // 2)
    i2 = jnp.stack([2 * idx_flat, 2 * idx_flat + 1], axis=1).reshape(-1)
    g = _sc_gather_rows(t2, i2)
    return g.reshape(idx_flat.shape[0], hdim)


def _sc_gather_halves(table, idx_flat):
    v, hdim = table.shape
    t2 = table.reshape(2 * v, hdim // 2)
    i2 = jnp.concatenate([2 * idx_flat, 2 * idx_flat + 1])
    g = _sc_gather_rows(t2, i2)
    return g.reshape(2, idx_flat.shape[0], hdim // 2)


def _sc_gather_rows(table, idx_flat):
    n = idx_flat.shape[0]
    assert n % GW == 0
    idx2 = idx_flat.reshape(1, n)
    mesh = plsc.VectorSubcoreMesh(core_axis_name="core", subcore_axis_name="subcore")

    @pl.kernel(
        out_type=jax.ShapeDtypeStruct((n, table.shape[1]), table.dtype),
        mesh=mesh,
    )
    def k(tab_hbm, i_hbm, o_hbm):
        def body(i_vmem, o_vmem):
            pltpu.sync_copy(tab_hbm.at[i_vmem.at[0]], o_vmem)

        pltpu.emit_pipeline(
            body,
            grid=(n // GW,),
            in_specs=[pl.BlockSpec((1, GW), index_map=lambda i: (0, i))],
            out_specs=[pl.BlockSpec((GW, table.shape[1]), index_map=lambda i: (i, 0))],
            core_axis_name=("core", "subcore"),
            dimension_semantics=(pltpu.PARALLEL,),
        )(i_hbm, o_hbm)

    return k(table, idx2)


def _ggnn_kernel(wlo_ref, whi_ref, adj_ref, Win, Wout, Wr, Wz, Wh,
                 bin_, bout, br, bz, bh, aW, ab, av, avb, out_ref):
    acc_lo = wlo_ref[0, 0]
    acc_hi = whi_ref[0, 0]
    for k in range(1, W_LEN):
        acc_lo = acc_lo + wlo_ref[0, k]
        acc_hi = acc_hi + whi_ref[0, k]
    st = jnp.concatenate([acc_lo, acc_hi], axis=1) * (1.0 / W_LEN)

    ain16 = [adj_ref[e, :, :N_NODE].astype(BF16) for e in range(BB)]
    aout16 = [adj_ref[e, :, N_NODE:].astype(BF16) for e in range(BB)]

    for _step in range(5):
        st16 = st.astype(BF16)
        s_in = jnp.dot(st16, Win[...], preferred_element_type=F32) + bin_[...]
        s_out = jnp.dot(st16, Wout[...], preferred_element_type=F32) + bout[...]
        si16 = s_in.astype(BF16)
        so16 = s_out.astype(BF16)
        ains, aouts = [], []
        for e in range(BB):
            sl = slice(e * N_NODE, (e + 1) * N_NODE)
            ains.append(jnp.dot(ain16[e], si16[sl], preferred_element_type=F32))
            aouts.append(jnp.dot(aout16[e], so16[sl], preferred_element_type=F32))
        a_in = jnp.concatenate(ains, axis=0)
        a_out = jnp.concatenate(aouts, axis=0)
        ai16 = a_in.astype(BF16)
        ao16 = a_out.astype(BF16)

        def gate(Wg, bg):
            return (jnp.dot(ai16, Wg[0:H], preferred_element_type=F32)
                    + jnp.dot(ao16, Wg[H:2 * H], preferred_element_type=F32)
                    + jnp.dot(st16, Wg[2 * H:3 * H], preferred_element_type=F32)
                    + bg[...])

        r = jax.nn.sigmoid(gate(Wr, br))
        z = jax.nn.sigmoid(gate(Wz, bz))
        rs16 = (r * st).astype(BF16)
        h_hat = jnp.tanh(jnp.dot(ai16, Wh[0:H], preferred_element_type=F32)
                         + jnp.dot(ao16, Wh[H:2 * H], preferred_element_type=F32)
                         + jnp.dot(rs16, Wh[2 * H:3 * H], preferred_element_type=F32)
                         + bh[...])
        st = st + z * (h_hat - st)

    st16 = st.astype(BF16)
    sa = jnp.tanh(jnp.dot(st16, aW[...], preferred_element_type=F32) + ab[...])
    sc = jnp.dot(sa.astype(BF16), av[...], preferred_element_type=F32) + avb[...]
    sc3 = sc.reshape(BB, N_NODE, 1)
    m = jnp.max(sc3, axis=1, keepdims=True)
    ew = jnp.exp(sc3 - m)
    wgt = ew / jnp.sum(ew, axis=1, keepdims=True)
    st3 = st.reshape(BB, N_NODE, H)
    pooled = jnp.sum(st3 * wgt, axis=1)
    t = jnp.tanh(pooled)
    nrm2 = jnp.sum(t * t, axis=1, keepdims=True)
    inv = 1.0 / jnp.maximum(jnp.sqrt(nrm2), 1e-8)
    out_ref[0] = t * inv


def _desc_loss_kernel(dlo_ref, dhi_ref, len_ref, Wi, Whh, lb, a2W, a2b, a2v,
                      a2vb, cT, out_ref, hs_ref, scores_ref):
    h = jnp.zeros((B, H), F32)
    c = jnp.zeros((B, H), F32)
    for t in range(DLEN):
        sl = slice(t * B, (t + 1) * B)
        x_t = jnp.concatenate([dlo_ref[0, sl], dhi_ref[0, sl]], axis=1)
        g = (jnp.dot(x_t.astype(BF16), Wi[...], preferred_element_type=F32)
             + jnp.dot(h.astype(BF16), Whh[...], preferred_element_type=F32)
             + lb[...])
        i_g = jax.nn.sigmoid(g[:, 0:H])
        f_g = jax.nn.sigmoid(g[:, H:2 * H])
        gg = jnp.tanh(g[:, 2 * H:3 * H])
        o_g = jax.nn.sigmoid(g[:, 3 * H:4 * H])
        c = f_g * c + i_g * gg
        h = o_g * jnp.tanh(c)
        h16 = h.astype(BF16)
        hs_ref[sl, :] = h16
        sa = jnp.tanh(jnp.dot(h16, a2W[...], preferred_element_type=F32) + a2b[...])
        s_t = jnp.dot(sa.astype(BF16), a2v[...], preferred_element_type=F32) + a2vb[...]
        scores_ref[:, t:t + 1] = s_t

    lens = len_ref[...]
    tpos = jax.lax.broadcasted_iota(jnp.int32, (B, DLEN), 1)
    sc = jnp.where(tpos < lens, scores_ref[...], -1e9)
    m = jnp.max(sc, axis=1, keepdims=True)
    e = jnp.exp(sc - m)
    wgt = e / jnp.sum(e, axis=1, keepdims=True)

    pooled = jnp.zeros((B, H), F32)
    for t in range(DLEN):
        sl = slice(t * B, (t + 1) * B)
        pooled = pooled + wgt[:, t:t + 1] * hs_ref[sl, :].astype(F32)
    dr = jnp.tanh(pooled)
    nrm = jnp.sqrt(jnp.sum(dr * dr, axis=1, keepdims=True))
    a = dr * (1.0 / jnp.maximum(nrm, 1e-8))

    logits = jnp.dot(a.astype(BF16), cT[...], preferred_element_type=F32) * 20.0
    lm = jnp.max(logits, axis=1, keepdims=True)
    lse = jnp.log(jnp.sum(jnp.exp(logits - lm), axis=1, keepdims=True)) + lm
    ii = jax.lax.broadcasted_iota(jnp.int32, (B, B), 0)
    jj = jax.lax.broadcasted_iota(jnp.int32, (B, B), 1)
    diag = jnp.sum(jnp.where(ii == jj, logits, 0.0), axis=1, keepdims=True)
    out_ref[...] = -jnp.mean(diag - lse, keepdims=True)


def kernel(ir_anno, ir_adjmat, ir_node_mask, ir_word_mask, desc_anchor,
           desc_anchor_len, desc_neg, desc_neg_len, ir_word_emb, W_in, b_in,
           W_out, b_out, Wr, br, Wz, bz, Wh, bh, desc_emb, lstm_Wi, lstm_Wh,
           lstm_b, attn_W, attn_b, attn_v, attn_vb, attn2_W, attn2_b,
           attn2_v, attn2_vb):
    d_idx = jnp.transpose(desc_anchor, (1, 0)).reshape(-1).astype(jnp.int32)
    d_g = _sc_gather_halves(desc_emb, d_idx)

    r2 = lambda x: x.reshape(1, -1)
    NC = 8
    CB = B // NC
    idx_c = jnp.transpose(ir_anno.reshape(NC, CB, N_NODE, W_LEN),
                          (0, 3, 1, 2)).reshape(NC, -1).astype(jnp.int32)

    ggnn_weights = (
        W_in.astype(BF16), W_out.astype(BF16),
        Wr.astype(BF16), Wz.astype(BF16), Wh.astype(BF16),
        r2(b_in), r2(b_out), r2(br), r2(bz), r2(bh),
        attn_W.astype(BF16), r2(attn_b), attn_v.astype(BF16),
        attn_vb.reshape(1, 1),
    )
    grid = CB // BB
    adj_c = ir_adjmat.reshape(NC, CB, N_NODE, 2 * N_NODE)
    c_parts = []
    for c in range(NC):
        w_g = _sc_gather_halves(ir_word_emb, idx_c[c]).reshape(
            2, W_LEN, CB * N_NODE, H // 2)
        c_parts.append(pl.pallas_call(
            _ggnn_kernel,
            grid=(grid,),
            in_specs=[
                pl.BlockSpec((1, W_LEN, BB * N_NODE, H // 2),
                             lambda i: (0, 0, i, 0)),
                pl.BlockSpec((1, W_LEN, BB * N_NODE, H // 2),
                             lambda i: (1, 0, i, 0)),
                pl.BlockSpec((BB, N_NODE, 2 * N_NODE), lambda i: (i, 0, 0)),
                pl.BlockSpec((H, H), lambda i: (0, 0)),
                pl.BlockSpec((H, H), lambda i: (0, 0)),
                pl.BlockSpec((3 * H, H), lambda i: (0, 0)),
                pl.BlockSpec((3 * H, H), lambda i: (0, 0)),
                pl.BlockSpec((3 * H, H), lambda i: (0, 0)),
                pl.BlockSpec((1, H), lambda i: (0, 0)),
                pl.BlockSpec((1, H), lambda i: (0, 0)),
                pl.BlockSpec((1, H), lambda i: (0, 0)),
                pl.BlockSpec((1, H), lambda i: (0, 0)),
                pl.BlockSpec((1, H), lambda i: (0, 0)),
                pl.BlockSpec((H, H), lambda i: (0, 0)),
                pl.BlockSpec((1, H), lambda i: (0, 0)),
                pl.BlockSpec((H, 1), lambda i: (0, 0)),
                pl.BlockSpec((1, 1), lambda i: (0, 0)),
            ],
            out_specs=pl.BlockSpec((1, BB, H), lambda i: (i, 0, 0)),
            out_shape=jax.ShapeDtypeStruct((grid, BB, H), F32),
        )(w_g, w_g, adj_c[c], *ggnn_weights))

    c_repr = jnp.concatenate([p.reshape(CB, H) for p in c_parts], axis=0)
    cT = jnp.transpose(c_repr).astype(BF16)

    loss = pl.pallas_call(
        _desc_loss_kernel,
        grid=(1,),
        in_specs=[
            pl.BlockSpec((1, DLEN * B, H // 2), lambda i: (0, 0, 0)),
            pl.BlockSpec((1, DLEN * B, H // 2), lambda i: (1, 0, 0)),
            pl.BlockSpec((B, 1), lambda i: (0, 0)),
            pl.BlockSpec((H, 4 * H), lambda i: (0, 0)),
            pl.BlockSpec((H, 4 * H), lambda i: (0, 0)),
            pl.BlockSpec((1, 4 * H), lambda i: (0, 0)),
            pl.BlockSpec((H, H), lambda i: (0, 0)),
            pl.BlockSpec((1, H), lambda i: (0, 0)),
            pl.BlockSpec((H, 1), lambda i: (0, 0)),
            pl.BlockSpec((1, 1), lambda i: (0, 0)),
            pl.BlockSpec((H, B), lambda i: (0, 0)),
        ],
        out_specs=pl.BlockSpec((1, 1), lambda i: (0, 0)),
        out_shape=jax.ShapeDtypeStruct((1, 1), F32),
        scratch_shapes=[
            pltpu.VMEM((DLEN * B, H), BF16),
            pltpu.VMEM((B, DLEN), F32),
        ],
    )(
        d_g, d_g, desc_anchor_len.reshape(B, 1).astype(jnp.int32),
        lstm_Wi.astype(BF16), lstm_Wh.astype(BF16), r2(lstm_b),
        attn2_W.astype(BF16), r2(attn2_b), attn2_v.astype(BF16),
        attn2_vb.reshape(1, 1), cT,
    )

    return loss.reshape(())


# speedup vs baseline: 1.0208x; 1.0208x over previous
import jax
import jax.numpy as jnp
from jax.experimental import pallas as pl
from jax.experimental.pallas import tpu as pltpu
from jax.experimental.pallas import tpu_sc as plsc

B, N_NODE, W_LEN, DLEN = 128, 200, 10, 30
H = 512
F32 = jnp.float32
BF16 = jnp.bfloat16
BB = 2
GW = 128


def _sc_gather(table, idx_flat):
    v, hdim = table.shape
    t2 = table.reshape(2 * v, hdim // 2)
    i2 = jnp.stack([2 * idx_flat, 2 * idx_flat + 1], axis=1).reshape(-1)
    g = _sc_gather_rows(t2, i2)
    return g.reshape(idx_flat.shape[0], hdim)


def _sc_gather_halves(table, idx_flat):
    v, hdim = table.shape
    t2 = table.reshape(2 * v, hdim // 2)
    i2 = jnp.concatenate([2 * idx_flat, 2 * idx_flat + 1])
    g = _sc_gather_rows(t2, i2)
    return g.reshape(2, idx_flat.shape[0], hdim // 2)


def _sc_gather_rows(table, idx_flat):
    n = idx_flat.shape[0]
    assert n % GW == 0
    idx2 = idx_flat.reshape(1, n)
    mesh = plsc.VectorSubcoreMesh(core_axis_name="core", subcore_axis_name="subcore")

    @pl.kernel(
        out_type=jax.ShapeDtypeStruct((n, table.shape[1]), table.dtype),
        mesh=mesh,
    )
    def k(tab_hbm, i_hbm, o_hbm):
        def body(i_vmem, o_vmem):
            pltpu.sync_copy(tab_hbm.at[i_vmem.at[0]], o_vmem)

        pltpu.emit_pipeline(
            body,
            grid=(n // GW,),
            in_specs=[pl.BlockSpec((1, GW), index_map=lambda i: (0, i))],
            out_specs=[pl.BlockSpec((GW, table.shape[1]), index_map=lambda i: (i, 0))],
            core_axis_name=("core", "subcore"),
            dimension_semantics=(pltpu.PARALLEL,),
        )(i_hbm, o_hbm)

    return k(table, idx2)


def _ggnn_kernel(wlo_ref, whi_ref, adj_ref, Win, Wout, Wr, Wz, Wh,
                 bin_, bout, br, bz, bh, aW, ab, av, avb, out_ref):
    acc_lo = wlo_ref[0, 0]
    acc_hi = whi_ref[0, 0]
    for k in range(1, W_LEN):
        acc_lo = acc_lo + wlo_ref[0, k]
        acc_hi = acc_hi + whi_ref[0, k]
    st = jnp.concatenate([acc_lo, acc_hi], axis=1) * (1.0 / W_LEN)

    ain16 = [adj_ref[e, :, :N_NODE].astype(BF16) for e in range(BB)]
    aout16 = [adj_ref[e, :, N_NODE:].astype(BF16) for e in range(BB)]

    for _step in range(5):
        st16 = st.astype(BF16)
        s_in = jnp.dot(st16, Win[...], preferred_element_type=F32) + bin_[...]
        s_out = jnp.dot(st16, Wout[...], preferred_element_type=F32) + bout[...]
        si16 = s_in.astype(BF16)
        so16 = s_out.astype(BF16)
        ains, aouts = [], []
        for e in range(BB):
            sl = slice(e * N_NODE, (e + 1) * N_NODE)
            ains.append(jnp.dot(ain16[e], si16[sl], preferred_element_type=F32))
            aouts.append(jnp.dot(aout16[e], so16[sl], preferred_element_type=F32))
        a_in = jnp.concatenate(ains, axis=0)
        a_out = jnp.concatenate(aouts, axis=0)
        ai16 = a_in.astype(BF16)
        ao16 = a_out.astype(BF16)

        def gate(Wg, bg):
            return (jnp.dot(ai16, Wg[0:H], preferred_element_type=F32)
                    + jnp.dot(ao16, Wg[H:2 * H], preferred_element_type=F32)
                    + jnp.dot(st16, Wg[2 * H:3 * H], preferred_element_type=F32)
                    + bg[...])

        r = jax.nn.sigmoid(gate(Wr, br))
        z = jax.nn.sigmoid(gate(Wz, bz))
        rs16 = (r * st).astype(BF16)
        h_hat = jnp.tanh(jnp.dot(ai16, Wh[0:H], preferred_element_type=F32)
                         + jnp.dot(ao16, Wh[H:2 * H], preferred_element_type=F32)
                         + jnp.dot(rs16, Wh[2 * H:3 * H], preferred_element_type=F32)
                         + bh[...])
        st = st + z * (h_hat - st)

    st16 = st.astype(BF16)
    sa = jnp.tanh(jnp.dot(st16, aW[...], preferred_element_type=F32) + ab[...])
    sc = jnp.dot(sa.astype(BF16), av[...], preferred_element_type=F32) + avb[...]
    sc3 = sc.reshape(BB, N_NODE, 1)
    m = jnp.max(sc3, axis=1, keepdims=True)
    ew = jnp.exp(sc3 - m)
    wgt = ew / jnp.sum(ew, axis=1, keepdims=True)
    st3 = st.reshape(BB, N_NODE, H)
    pooled = jnp.sum(st3 * wgt, axis=1)
    t = jnp.tanh(pooled)
    nrm2 = jnp.sum(t * t, axis=1, keepdims=True)
    inv = 1.0 / jnp.maximum(jnp.sqrt(nrm2), 1e-8)
    out_ref[0] = t * inv


def _desc_loss_kernel(dlo_ref, dhi_ref, len_ref, Wi, Whh, lb, a2W, a2b, a2v,
                      a2vb, cT, out_ref, hs_ref, scores_ref):
    h = jnp.zeros((B, H), F32)
    c = jnp.zeros((B, H), F32)
    for t in range(DLEN):
        sl = slice(t * B, (t + 1) * B)
        x_t = jnp.concatenate([dlo_ref[0, sl], dhi_ref[0, sl]], axis=1)
        g = (jnp.dot(x_t.astype(BF16), Wi[...], preferred_element_type=F32)
             + jnp.dot(h.astype(BF16), Whh[...], preferred_element_type=F32)
             + lb[...])
        i_g = jax.nn.sigmoid(g[:, 0:H])
        f_g = jax.nn.sigmoid(g[:, H:2 * H])
        gg = jnp.tanh(g[:, 2 * H:3 * H])
        o_g = jax.nn.sigmoid(g[:, 3 * H:4 * H])
        c = f_g * c + i_g * gg
        h = o_g * jnp.tanh(c)
        h16 = h.astype(BF16)
        hs_ref[sl, :] = h16
        sa = jnp.tanh(jnp.dot(h16, a2W[...], preferred_element_type=F32) + a2b[...])
        s_t = jnp.dot(sa.astype(BF16), a2v[...], preferred_element_type=F32) + a2vb[...]
        scores_ref[:, t:t + 1] = s_t

    lens = len_ref[...]
    tpos = jax.lax.broadcasted_iota(jnp.int32, (B, DLEN), 1)
    sc = jnp.where(tpos < lens, scores_ref[...], -1e9)
    m = jnp.max(sc, axis=1, keepdims=True)
    e = jnp.exp(sc - m)
    wgt = e / jnp.sum(e, axis=1, keepdims=True)

    pooled = jnp.zeros((B, H), F32)
    for t in range(DLEN):
        sl = slice(t * B, (t + 1) * B)
        pooled = pooled + wgt[:, t:t + 1] * hs_ref[sl, :].astype(F32)
    dr = jnp.tanh(pooled)
    nrm = jnp.sqrt(jnp.sum(dr * dr, axis=1, keepdims=True))
    a = dr * (1.0 / jnp.maximum(nrm, 1e-8))

    logits = jnp.dot(a.astype(BF16), cT[...], preferred_element_type=F32) * 20.0
    lm = jnp.max(logits, axis=1, keepdims=True)
    lse = jnp.log(jnp.sum(jnp.exp(logits - lm), axis=1, keepdims=True)) + lm
    ii = jax.lax.broadcasted_iota(jnp.int32, (B, B), 0)
    jj = jax.lax.broadcasted_iota(jnp.int32, (B, B), 1)
    diag = jnp.sum(jnp.where(ii == jj, logits, 0.0), axis=1, keepdims=True)
    out_ref[...] = -jnp.mean(diag - lse, keepdims=True)


def kernel(ir_anno, ir_adjmat, ir_node_mask, ir_word_mask, desc_anchor,
           desc_anchor_len, desc_neg, desc_neg_len, ir_word_emb, W_in, b_in,
           W_out, b_out, Wr, br, Wz, bz, Wh, bh, desc_emb, lstm_Wi, lstm_Wh,
           lstm_b, attn_W, attn_b, attn_v, attn_vb, attn2_W, attn2_b,
           attn2_v, attn2_vb):
    d_idx = jnp.transpose(desc_anchor, (1, 0)).reshape(-1).astype(jnp.int32)
    d_g = _sc_gather_halves(desc_emb, d_idx)

    r2 = lambda x: x.reshape(1, -1)
    NC = 4
    CB = B // NC
    idx_c = jnp.transpose(ir_anno.reshape(NC, CB, N_NODE, W_LEN),
                          (0, 3, 1, 2)).reshape(NC, -1).astype(jnp.int32)

    ggnn_weights = (
        W_in.astype(BF16), W_out.astype(BF16),
        Wr.astype(BF16), Wz.astype(BF16), Wh.astype(BF16),
        r2(b_in), r2(b_out), r2(br), r2(bz), r2(bh),
        attn_W.astype(BF16), r2(attn_b), attn_v.astype(BF16),
        attn_vb.reshape(1, 1),
    )
    grid = CB // BB
    adj_c = ir_adjmat.reshape(NC, CB, N_NODE, 2 * N_NODE)
    c_parts = []
    for c in range(NC):
        w_g = _sc_gather_halves(ir_word_emb, idx_c[c]).reshape(
            2, W_LEN, CB * N_NODE, H // 2)
        c_parts.append(pl.pallas_call(
            _ggnn_kernel,
            grid=(grid,),
            in_specs=[
                pl.BlockSpec((1, W_LEN, BB * N_NODE, H // 2),
                             lambda i: (0, 0, i, 0)),
                pl.BlockSpec((1, W_LEN, BB * N_NODE, H // 2),
                             lambda i: (1, 0, i, 0)),
                pl.BlockSpec((BB, N_NODE, 2 * N_NODE), lambda i: (i, 0, 0)),
                pl.BlockSpec((H, H), lambda i: (0, 0)),
                pl.BlockSpec((H, H), lambda i: (0, 0)),
                pl.BlockSpec((3 * H, H), lambda i: (0, 0)),
                pl.BlockSpec((3 * H, H), lambda i: (0, 0)),
                pl.BlockSpec((3 * H, H), lambda i: (0, 0)),
                pl.BlockSpec((1, H), lambda i: (0, 0)),
                pl.BlockSpec((1, H), lambda i: (0, 0)),
                pl.BlockSpec((1, H), lambda i: (0, 0)),
                pl.BlockSpec((1, H), lambda i: (0, 0)),
                pl.BlockSpec((1, H), lambda i: (0, 0)),
                pl.BlockSpec((H, H), lambda i: (0, 0)),
                pl.BlockSpec((1, H), lambda i: (0, 0)),
                pl.BlockSpec((H, 1), lambda i: (0, 0)),
                pl.BlockSpec((1, 1), lambda i: (0, 0)),
            ],
            out_specs=pl.BlockSpec((1, BB, H), lambda i: (i, 0, 0)),
            out_shape=jax.ShapeDtypeStruct((grid, BB, H), F32),
        )(w_g, w_g, adj_c[c], *ggnn_weights))

    c_repr = jnp.concatenate([p.reshape(CB, H) for p in c_parts], axis=0)
    cT = jnp.transpose(c_repr).astype(BF16)

    loss = pl.pallas_call(
        _desc_loss_kernel,
        grid=(1,),
        in_specs=[
            pl.BlockSpec((1, DLEN * B, H // 2), lambda i: (0, 0, 0)),
            pl.BlockSpec((1, DLEN * B, H // 2), lambda i: (1, 0, 0)),
            pl.BlockSpec((B, 1), lambda i: (0, 0)),
            pl.BlockSpec((H, 4 * H), lambda i: (0, 0)),
            pl.BlockSpec((H, 4 * H), lambda i: (0, 0)),
            pl.BlockSpec((1, 4 * H), lambda i: (0, 0)),
            pl.BlockSpec((H, H), lambda i: (0, 0)),
            pl.BlockSpec((1, H), lambda i: (0, 0)),
            pl.BlockSpec((H, 1), lambda i: (0, 0)),
            pl.BlockSpec((1, 1), lambda i: (0, 0)),
            pl.BlockSpec((H, B), lambda i: (0, 0)),
        ],
        out_specs=pl.BlockSpec((1, 1), lambda i: (0, 0)),
        out_shape=jax.ShapeDtypeStruct((1, 1), F32),
        scratch_shapes=[
            pltpu.VMEM((DLEN * B, H), BF16),
            pltpu.VMEM((B, DLEN), F32),
        ],
    )(
        d_g, d_g, desc_anchor_len.reshape(B, 1).astype(jnp.int32),
        lstm_Wi.astype(BF16), lstm_Wh.astype(BF16), r2(lstm_b),
        attn2_W.astype(BF16), r2(attn2_b), attn2_v.astype(BF16),
        attn2_vb.reshape(1, 1), cT,
    )

    return loss.reshape(())
